# baseline (speedup 1.0000x reference)
import functools

import jax
import jax.numpy as jnp
from jax import lax
from jax.experimental import pallas as pl
from jax.experimental.pallas import tpu as pltpu
from jax.experimental.pallas import tpu_sc as plsc

NUM_PROPOSALS = 1000
NMS_THR = 0.5
SCORE_THR = 0.05
M = 6000
MPAD = 6080
L = 16
UNROLL = 4
NW = 16
B = 32
CPB = B // NW
BPAD = B + L
KSZ = 1088
KOUT = 1024

_GATHER_DNUMS = lax.GatherDimensionNumbers(
    offset_dims=(), collapsed_slice_dims=(0,), start_index_map=(0,)
)


def _broadcast_lane(vec, lane):
    idx = jnp.full((L, 1), lane, jnp.int32)
    return lax.gather(
        vec, idx, _GATHER_DNUMS, (1,),
        mode=lax.GatherScatterMode.PROMISE_IN_BOUNDS,
    )


def _any_hit(acc):
    return plsc.all_reduce_population_count(acc > NMS_THR)[0] > 0


def _nms_body(x1h, y1h, x2h, y2h, ssh,
              ox1h, oy1h, ox2h, oy2h, osch,
              x1v, y1v, x2v, y2v, ssv,
              ox1, oy1, ox2, oy2, osc, kav,
              sbuf, fstage, istage,
              skx1, sky1, skx2, sky2, ska, ssup, smeta):
    cid = lax.axis_index("c")
    wid = lax.axis_index("s")

    @pl.when(cid == 0)
    def _():
        pltpu.sync_copy(x1h, x1v)
        pltpu.sync_copy(y1h, y1v)
        pltpu.sync_copy(x2h, x2v)
        pltpu.sync_copy(y2h, y2v)
        pltpu.sync_copy(ssh, ssv)

        zeros16 = jnp.zeros((L,), jnp.float32)

        def zero_block(c, carry):
            ox1[pl.ds(c * L, L)] = zeros16
            oy1[pl.ds(c * L, L)] = zeros16
            ox2[pl.ds(c * L, L)] = zeros16
            oy2[pl.ds(c * L, L)] = zeros16
            osc[pl.ds(c * L, L)] = zeros16
            kav[pl.ds(c * L, L)] = zeros16
            return carry

        lax.fori_loop(0, KSZ // L, zero_block, 0)

        lane_ids = lax.broadcasted_iota(jnp.int32, (L,), 0)
        mask0 = lane_ids == 0

        def load_cand(i):
            base = (i // L) * L
            lane = i - base
            cx1 = _broadcast_lane(x1v[pl.ds(base, L)], lane)
            cy1 = _broadcast_lane(y1v[pl.ds(base, L)], lane)
            cx2 = _broadcast_lane(x2v[pl.ds(base, L)], lane)
            cy2 = _broadcast_lane(y2v[pl.ds(base, L)], lane)
            ca = jnp.maximum(cx2 - cx1, 0.0) * jnp.maximum(cy2 - cy1, 0.0)
            return cx1, cy1, cx2, cy2, ca

        def append(box, kidx_scalar, score):
            cx1, cy1, cx2, cy2, ca = box
            kidx = jnp.full((L,), kidx_scalar, jnp.int32)
            plsc.store_scatter(ox1, [kidx], cx1, mask=mask0)
            plsc.store_scatter(oy1, [kidx], cy1, mask=mask0)
            plsc.store_scatter(ox2, [kidx], cx2, mask=mask0)
            plsc.store_scatter(oy2, [kidx], cy2, mask=mask0)
            plsc.store_scatter(osc, [kidx], jnp.full((L,), score), mask=mask0)
            plsc.store_scatter(kav, [kidx], ca, mask=mask0)

        def pair_flags(ia, k0):
            ax1, ay1, ax2, ay2, aa = load_cand(ia)
            bx1, by1, bx2, by2, ba = load_cand(ia + 1)
            nch = (k0 + (UNROLL * L - 1)) // (UNROLL * L)

            def chunk(c, accs):
                acca, accb = accs
                for u in range(UNROLL):
                    off = c * (UNROLL * L) + u * L
                    kx1 = ox1[pl.ds(off, L)]
                    ky1 = oy1[pl.ds(off, L)]
                    kx2 = ox2[pl.ds(off, L)]
                    ky2 = oy2[pl.ds(off, L)]
                    ka = kav[pl.ds(off, L)]
                    iwa = jnp.maximum(jnp.minimum(ax2, kx2) - jnp.maximum(ax1, kx1), 0.0)
                    iha = jnp.maximum(jnp.minimum(ay2, ky2) - jnp.maximum(ay1, ky1), 0.0)
                    intera = iwa * iha
                    acca = jnp.maximum(acca, intera / jnp.maximum(aa + ka - intera, 1e-9))
                    iwb = jnp.maximum(jnp.minimum(bx2, kx2) - jnp.maximum(bx1, kx1), 0.0)
                    ihb = jnp.maximum(jnp.minimum(by2, ky2) - jnp.maximum(by1, ky1), 0.0)
                    interb = iwb * ihb
                    accb = jnp.maximum(accb, interb / jnp.maximum(ba + ka - interb, 1e-9))
                return acca, accb

            acca, accb = lax.fori_loop(0, nch, chunk, (zeros16, zeros16))
            fa = jnp.where(_any_hit(acca), 1.0, 0.0)
            fb = jnp.where(_any_hit(accb), 1.0, 0.0)
            return fa, fb

        def cond(state):
            b, kept, s = state
            return (b * B < M) & (kept < NUM_PROPOSALS) & (s > SCORE_THR)

        def body(state):
            b, kept, s = state
            i0 = b * B
            k0 = kept
            k0c = k0 // L
            k0a = k0c * L

            flags = zeros16
            for q in range(CPB // 2):
                ia = i0 + wid * CPB + 2 * q
                fa, fb = pair_flags(ia, k0)
                flags = (flags
                         + jnp.where(lane_ids == 2 * q, fa, 0.0)
                         + jnp.where(lane_ids == 2 * q + 1, fb, 0.0))
            fstage[...] = flags
            pltpu.sync_copy(fstage, ssup.at[pl.ds(wid * L, L)])
            plsc.subcore_barrier()

            @pl.when(wid == 0)
            def _serial():
                pltpu.sync_copy(ssup, sbuf)

                def jbody(j, kj):
                    idx = i0 + j
                    base = (idx // L) * L
                    sj = _broadcast_lane(ssv[pl.ds(base, L)], idx - base)[0]
                    w = j // CPB
                    fj = _broadcast_lane(sbuf[pl.ds(w * L, L)], j - w * CPB)[0]
                    alive = ((sj > SCORE_THR) & (fj < 0.5)
                             & (kj < NUM_PROPOSALS))

                    def live():
                        box = load_cand(idx)
                        cx1, cy1, cx2, cy2, ca = box
                        cend = (kj + (L - 1)) // L

                        def ch1(c, acc):
                            kx1 = ox1[pl.ds(c * L, L)]
                            ky1 = oy1[pl.ds(c * L, L)]
                            kx2 = ox2[pl.ds(c * L, L)]
                            ky2 = oy2[pl.ds(c * L, L)]
                            ka = kav[pl.ds(c * L, L)]
                            iw = jnp.maximum(jnp.minimum(cx2, kx2) - jnp.maximum(cx1, kx1), 0.0)
                            ih = jnp.maximum(jnp.minimum(cy2, ky2) - jnp.maximum(cy1, ky1), 0.0)
                            inter = iw * ih
                            return jnp.maximum(acc, inter / jnp.maximum(ca + ka - inter, 1e-9))

                        acc = lax.fori_loop(k0c, cend, ch1, zeros16)
                        supp = _any_hit(acc)

                        @pl.when(jnp.logical_not(supp))
                        def _():
                            append(box, kj, sj)

                        return kj + jnp.where(supp, 0, 1)

                    return lax.cond(alive, live, lambda: kj)

                kept_new = lax.fori_loop(0, B, jbody, kept)
                pltpu.sync_copy(ox1.at[pl.ds(k0a, BPAD)], skx1.at[pl.ds(k0a, BPAD)])
                pltpu.sync_copy(oy1.at[pl.ds(k0a, BPAD)], sky1.at[pl.ds(k0a, BPAD)])
                pltpu.sync_copy(ox2.at[pl.ds(k0a, BPAD)], skx2.at[pl.ds(k0a, BPAD)])
                pltpu.sync_copy(oy2.at[pl.ds(k0a, BPAD)], sky2.at[pl.ds(k0a, BPAD)])
                pltpu.sync_copy(kav.at[pl.ds(k0a, BPAD)], ska.at[pl.ds(k0a, BPAD)])
                istage[...] = jnp.full((L,), kept_new, jnp.int32)
                pltpu.sync_copy(istage, smeta)

            plsc.subcore_barrier()

            pltpu.sync_copy(smeta, istage)
            kept2 = istage[...][0]

            @pl.when(wid != 0)
            def _pull():
                pltpu.sync_copy(skx1.at[pl.ds(k0a, BPAD)], ox1.at[pl.ds(k0a, BPAD)])
                pltpu.sync_copy(sky1.at[pl.ds(k0a, BPAD)], oy1.at[pl.ds(k0a, BPAD)])
                pltpu.sync_copy(skx2.at[pl.ds(k0a, BPAD)], ox2.at[pl.ds(k0a, BPAD)])
                pltpu.sync_copy(sky2.at[pl.ds(k0a, BPAD)], oy2.at[pl.ds(k0a, BPAD)])
                pltpu.sync_copy(ska.at[pl.ds(k0a, BPAD)], kav.at[pl.ds(k0a, BPAD)])

            b2 = b + 1
            sn = ssv[pl.ds(b2 * B, L)][0]
            return (b2, kept2, sn)

        s0 = ssv[pl.ds(0, L)][0]
        lax.while_loop(cond, body, (jnp.int32(0), jnp.int32(0), s0))

        @pl.when(wid == 0)
        def _out():
            pltpu.sync_copy(ox1.at[pl.ds(0, KOUT)], ox1h)
            pltpu.sync_copy(oy1.at[pl.ds(0, KOUT)], oy1h)
            pltpu.sync_copy(ox2.at[pl.ds(0, KOUT)], ox2h)
            pltpu.sync_copy(oy2.at[pl.ds(0, KOUT)], oy2h)
            pltpu.sync_copy(osc.at[pl.ds(0, KOUT)], osch)


_f32 = jnp.float32
_i32 = jnp.int32
_out1k = jax.ShapeDtypeStruct((KOUT,), _f32)


@functools.cache
def _nms_call():
    return functools.partial(
        pl.kernel,
        out_type=(_out1k,) * 5,
        mesh=plsc.VectorSubcoreMesh(core_axis_name="c", subcore_axis_name="s"),
        scratch_types=(
            [pltpu.VMEM((MPAD,), _f32)] * 5
            + [pltpu.VMEM((KSZ,), _f32)] * 6
            + [pltpu.VMEM((NW * L,), _f32),
               pltpu.VMEM((L,), _f32),
               pltpu.VMEM((L,), _i32)]
            + [pltpu.VMEM_SHARED((KSZ,), _f32)] * 5
            + [pltpu.VMEM_SHARED((NW * L,), _f32),
               pltpu.VMEM_SHARED((L,), _i32)]
        ),
        compiler_params=pltpu.CompilerParams(needs_layout_passes=False),
    )(_nms_body)


@jax.jit
def kernel(cache_boxes, proposal_boxes, proposal_logits):
    scores_new = jax.nn.sigmoid(proposal_logits)
    merged_boxes = jnp.concatenate([cache_boxes[:, :4], proposal_boxes], axis=0)
    merged_scores = jnp.concatenate([cache_boxes[:, 4], scores_new], axis=0)
    eff = jnp.where(merged_scores > SCORE_THR, merged_scores, -jnp.inf)
    order = jnp.argsort(-eff)
    sb = merged_boxes[order]
    ss = eff[order]
    pad = MPAD - M
    x1 = jnp.pad(sb[:, 0], (0, pad))
    y1 = jnp.pad(sb[:, 1], (0, pad))
    x2 = jnp.pad(sb[:, 2], (0, pad))
    y2 = jnp.pad(sb[:, 3], (0, pad))
    ssp = jnp.pad(ss, (0, pad), constant_values=-jnp.inf)
    ox1, oy1, ox2, oy2, osc = _nms_call()(x1, y1, x2, y2, ssp)
    out = jnp.stack([ox1, oy1, ox2, oy2, osc], axis=1)
    return out[:NUM_PROPOSALS]


# speedup vs baseline: 1.3178x; 1.3178x over previous
import functools

import jax
import jax.numpy as jnp
from jax import lax
from jax.experimental import pallas as pl
from jax.experimental.pallas import tpu as pltpu
from jax.experimental.pallas import tpu_sc as plsc

NUM_PROPOSALS = 1000
NMS_THR = 0.5
SCORE_THR = 0.05
M = 6000
MPAD = 6080
L = 16
UNROLL = 4
NW = 16
B = 64
CPB = B // NW
BPAD = B + L
KSZ = 1088
KOUT = 1024

_GATHER_DNUMS = lax.GatherDimensionNumbers(
    offset_dims=(), collapsed_slice_dims=(0,), start_index_map=(0,)
)


def _broadcast_lane(vec, lane):
    idx = jnp.full((L, 1), lane, jnp.int32)
    return lax.gather(
        vec, idx, _GATHER_DNUMS, (1,),
        mode=lax.GatherScatterMode.PROMISE_IN_BOUNDS,
    )


def _any_hit(acc):
    return plsc.all_reduce_population_count(acc > NMS_THR)[0] > 0


def _nms_body(x1h, y1h, x2h, y2h, ssh,
              ox1h, oy1h, ox2h, oy2h, osch,
              x1v, y1v, x2v, y2v, ssv,
              ox1, oy1, ox2, oy2, osc, kav,
              sbuf, fstage, istage,
              skx1, sky1, skx2, sky2, ska, ssup, smeta):
    cid = lax.axis_index("c")
    wid = lax.axis_index("s")

    @pl.when(cid == 0)
    def _():
        pltpu.sync_copy(x1h, x1v)
        pltpu.sync_copy(y1h, y1v)
        pltpu.sync_copy(x2h, x2v)
        pltpu.sync_copy(y2h, y2v)
        pltpu.sync_copy(ssh, ssv)

        zeros16 = jnp.zeros((L,), jnp.float32)

        def zero_block(c, carry):
            ox1[pl.ds(c * L, L)] = zeros16
            oy1[pl.ds(c * L, L)] = zeros16
            ox2[pl.ds(c * L, L)] = zeros16
            oy2[pl.ds(c * L, L)] = zeros16
            osc[pl.ds(c * L, L)] = zeros16
            kav[pl.ds(c * L, L)] = zeros16
            return carry

        lax.fori_loop(0, KSZ // L, zero_block, 0)

        lane_ids = lax.broadcasted_iota(jnp.int32, (L,), 0)
        mask0 = lane_ids == 0

        def load_cand(i):
            base = (i // L) * L
            lane = i - base
            cx1 = _broadcast_lane(x1v[pl.ds(base, L)], lane)
            cy1 = _broadcast_lane(y1v[pl.ds(base, L)], lane)
            cx2 = _broadcast_lane(x2v[pl.ds(base, L)], lane)
            cy2 = _broadcast_lane(y2v[pl.ds(base, L)], lane)
            ca = jnp.maximum(cx2 - cx1, 0.0) * jnp.maximum(cy2 - cy1, 0.0)
            return cx1, cy1, cx2, cy2, ca

        def append(box, kidx_scalar, score):
            cx1, cy1, cx2, cy2, ca = box
            kidx = jnp.full((L,), kidx_scalar, jnp.int32)
            plsc.store_scatter(ox1, [kidx], cx1, mask=mask0)
            plsc.store_scatter(oy1, [kidx], cy1, mask=mask0)
            plsc.store_scatter(ox2, [kidx], cx2, mask=mask0)
            plsc.store_scatter(oy2, [kidx], cy2, mask=mask0)
            plsc.store_scatter(osc, [kidx], jnp.full((L,), score), mask=mask0)
            plsc.store_scatter(kav, [kidx], ca, mask=mask0)

        def pair_flags(ia, i0, p, k0):
            ax1, ay1, ax2, ay2, aa = load_cand(ia)
            bx1, by1, bx2, by2, ba = load_cand(ia + 1)
            nch = (k0 + (UNROLL * L - 1)) // (UNROLL * L)

            def chunk(c, accs):
                acca, accb = accs
                for u in range(UNROLL):
                    off = c * (UNROLL * L) + u * L
                    kx1 = ox1[pl.ds(off, L)]
                    ky1 = oy1[pl.ds(off, L)]
                    kx2 = ox2[pl.ds(off, L)]
                    ky2 = oy2[pl.ds(off, L)]
                    ka = kav[pl.ds(off, L)]
                    iwa = jnp.maximum(jnp.minimum(ax2, kx2) - jnp.maximum(ax1, kx1), 0.0)
                    iha = jnp.maximum(jnp.minimum(ay2, ky2) - jnp.maximum(ay1, ky1), 0.0)
                    intera = iwa * iha
                    acca = jnp.maximum(acca, intera / jnp.maximum(aa + ka - intera, 1e-9))
                    iwb = jnp.maximum(jnp.minimum(bx2, kx2) - jnp.maximum(bx1, kx1), 0.0)
                    ihb = jnp.maximum(jnp.minimum(by2, ky2) - jnp.maximum(by1, ky1), 0.0)
                    interb = iwb * ihb
                    accb = jnp.maximum(accb, interb / jnp.maximum(ba + ka - interb, 1e-9))
                return acca, accb

            acca, accb = lax.fori_loop(0, nch, chunk, (zeros16, zeros16))

            def inblk(t, accs):
                da, db = accs
                off = i0 + t * L
                kx1 = x1v[pl.ds(off, L)]
                ky1 = y1v[pl.ds(off, L)]
                kx2 = x2v[pl.ds(off, L)]
                ky2 = y2v[pl.ds(off, L)]
                ka = (jnp.maximum(kx2 - kx1, 0.0)
                      * jnp.maximum(ky2 - ky1, 0.0))
                lim = p - t * L
                iwa = jnp.maximum(jnp.minimum(ax2, kx2) - jnp.maximum(ax1, kx1), 0.0)
                iha = jnp.maximum(jnp.minimum(ay2, ky2) - jnp.maximum(ay1, ky1), 0.0)
                intera = iwa * iha
                ioua = intera / jnp.maximum(aa + ka - intera, 1e-9)
                da = jnp.maximum(da, jnp.where(lane_ids < lim, ioua, 0.0))
                iwb = jnp.maximum(jnp.minimum(bx2, kx2) - jnp.maximum(bx1, kx1), 0.0)
                ihb = jnp.maximum(jnp.minimum(by2, ky2) - jnp.maximum(by1, ky1), 0.0)
                interb = iwb * ihb
                ioub = interb / jnp.maximum(ba + ka - interb, 1e-9)
                db = jnp.maximum(db, jnp.where(lane_ids < lim + 1, ioub, 0.0))
                return da, db

            da, db = lax.fori_loop(0, p // L + 1, inblk, (zeros16, zeros16))
            fa = (jnp.where(_any_hit(acca), 2.0, 0.0)
                  + jnp.where(_any_hit(da), 1.0, 0.0))
            fb = (jnp.where(_any_hit(accb), 2.0, 0.0)
                  + jnp.where(_any_hit(db), 1.0, 0.0))
            return fa, fb

        def cond(state):
            b, kept, s = state
            return (b * B < M) & (kept < NUM_PROPOSALS) & (s > SCORE_THR)

        def body(state):
            b, kept, s = state
            i0 = b * B
            k0 = kept
            k0c = k0 // L
            k0a = k0c * L

            flags = zeros16
            for q in range(CPB // 2):
                p = wid * CPB + 2 * q
                fa, fb = pair_flags(i0 + p, i0, p, k0)
                flags = (flags
                         + jnp.where(lane_ids == 2 * q, fa, 0.0)
                         + jnp.where(lane_ids == 2 * q + 1, fb, 0.0))
            fstage[...] = flags
            pltpu.sync_copy(fstage, ssup.at[pl.ds(wid * L, L)])
            plsc.subcore_barrier()

            @pl.when(wid == 0)
            def _serial():
                pltpu.sync_copy(ssup, sbuf)

                def jbody(j, kj):
                    idx = i0 + j
                    base = (idx // L) * L
                    sj = _broadcast_lane(ssv[pl.ds(base, L)], idx - base)[0]
                    w = j // CPB
                    fj = _broadcast_lane(sbuf[pl.ds(w * L, L)], j - w * CPB)[0]
                    alive = ((sj > SCORE_THR) & (fj < 1.5)
                             & (kj < NUM_PROPOSALS))

                    def live():
                        box = load_cand(idx)
                        cx1, cy1, cx2, cy2, ca = box

                        def dirty_test():
                            cend = (kj + (L - 1)) // L

                            def ch1(c, acc):
                                kx1 = ox1[pl.ds(c * L, L)]
                                ky1 = oy1[pl.ds(c * L, L)]
                                kx2 = ox2[pl.ds(c * L, L)]
                                ky2 = oy2[pl.ds(c * L, L)]
                                ka = kav[pl.ds(c * L, L)]
                                iw = jnp.maximum(jnp.minimum(cx2, kx2) - jnp.maximum(cx1, kx1), 0.0)
                                ih = jnp.maximum(jnp.minimum(cy2, ky2) - jnp.maximum(cy1, ky1), 0.0)
                                inter = iw * ih
                                return jnp.maximum(acc, inter / jnp.maximum(ca + ka - inter, 1e-9))

                            acc = lax.fori_loop(k0c, cend, ch1, zeros16)
                            return _any_hit(acc)

                        supp = lax.cond(fj > 0.5, dirty_test,
                                        lambda: jnp.bool_(False))

                        @pl.when(jnp.logical_not(supp))
                        def _():
                            append(box, kj, sj)

                        return kj + jnp.where(supp, 0, 1)

                    return lax.cond(alive, live, lambda: kj)

                kept_new = lax.fori_loop(0, B, jbody, kept)
                pltpu.sync_copy(ox1.at[pl.ds(k0a, BPAD)], skx1.at[pl.ds(k0a, BPAD)])
                pltpu.sync_copy(oy1.at[pl.ds(k0a, BPAD)], sky1.at[pl.ds(k0a, BPAD)])
                pltpu.sync_copy(ox2.at[pl.ds(k0a, BPAD)], skx2.at[pl.ds(k0a, BPAD)])
                pltpu.sync_copy(oy2.at[pl.ds(k0a, BPAD)], sky2.at[pl.ds(k0a, BPAD)])
                pltpu.sync_copy(kav.at[pl.ds(k0a, BPAD)], ska.at[pl.ds(k0a, BPAD)])
                istage[...] = jnp.full((L,), kept_new, jnp.int32)
                pltpu.sync_copy(istage, smeta)

            plsc.subcore_barrier()

            pltpu.sync_copy(smeta, istage)
            kept2 = istage[...][0]

            @pl.when(wid != 0)
            def _pull():
                pltpu.sync_copy(skx1.at[pl.ds(k0a, BPAD)], ox1.at[pl.ds(k0a, BPAD)])
                pltpu.sync_copy(sky1.at[pl.ds(k0a, BPAD)], oy1.at[pl.ds(k0a, BPAD)])
                pltpu.sync_copy(skx2.at[pl.ds(k0a, BPAD)], ox2.at[pl.ds(k0a, BPAD)])
                pltpu.sync_copy(sky2.at[pl.ds(k0a, BPAD)], oy2.at[pl.ds(k0a, BPAD)])
                pltpu.sync_copy(ska.at[pl.ds(k0a, BPAD)], kav.at[pl.ds(k0a, BPAD)])

            b2 = b + 1
            sn = ssv[pl.ds(b2 * B, L)][0]
            return (b2, kept2, sn)

        s0 = ssv[pl.ds(0, L)][0]
        lax.while_loop(cond, body, (jnp.int32(0), jnp.int32(0), s0))

        @pl.when(wid == 0)
        def _out():
            pltpu.sync_copy(ox1.at[pl.ds(0, KOUT)], ox1h)
            pltpu.sync_copy(oy1.at[pl.ds(0, KOUT)], oy1h)
            pltpu.sync_copy(ox2.at[pl.ds(0, KOUT)], ox2h)
            pltpu.sync_copy(oy2.at[pl.ds(0, KOUT)], oy2h)
            pltpu.sync_copy(osc.at[pl.ds(0, KOUT)], osch)


_f32 = jnp.float32
_i32 = jnp.int32
_out1k = jax.ShapeDtypeStruct((KOUT,), _f32)


@functools.cache
def _nms_call():
    return functools.partial(
        pl.kernel,
        out_type=(_out1k,) * 5,
        mesh=plsc.VectorSubcoreMesh(core_axis_name="c", subcore_axis_name="s"),
        scratch_types=(
            [pltpu.VMEM((MPAD,), _f32)] * 5
            + [pltpu.VMEM((KSZ,), _f32)] * 6
            + [pltpu.VMEM((NW * L,), _f32),
               pltpu.VMEM((L,), _f32),
               pltpu.VMEM((L,), _i32)]
            + [pltpu.VMEM_SHARED((KSZ,), _f32)] * 5
            + [pltpu.VMEM_SHARED((NW * L,), _f32),
               pltpu.VMEM_SHARED((L,), _i32)]
        ),
        compiler_params=pltpu.CompilerParams(needs_layout_passes=False),
    )(_nms_body)


@jax.jit
def kernel(cache_boxes, proposal_boxes, proposal_logits):
    scores_new = jax.nn.sigmoid(proposal_logits)
    merged_boxes = jnp.concatenate([cache_boxes[:, :4], proposal_boxes], axis=0)
    merged_scores = jnp.concatenate([cache_boxes[:, 4], scores_new], axis=0)
    eff = jnp.where(merged_scores > SCORE_THR, merged_scores, -jnp.inf)
    order = jnp.argsort(-eff)
    sb = merged_boxes[order]
    ss = eff[order]
    pad = MPAD - M
    x1 = jnp.pad(sb[:, 0], (0, pad))
    y1 = jnp.pad(sb[:, 1], (0, pad))
    x2 = jnp.pad(sb[:, 2], (0, pad))
    y2 = jnp.pad(sb[:, 3], (0, pad))
    ssp = jnp.pad(ss, (0, pad), constant_values=-jnp.inf)
    ox1, oy1, ox2, oy2, osc = _nms_call()(x1, y1, x2, y2, ssp)
    out = jnp.stack([ox1, oy1, ox2, oy2, osc], axis=1)
    return out[:NUM_PROPOSALS]
